# baseline (speedup 1.0000x reference)
import jax
import jax.numpy as jnp
from jax import lax
from jax.experimental import pallas as pl
from jax.experimental.pallas import tpu as pltpu
from jax.experimental.pallas import tpu_sc as plsc

M, J, I, L, W = 16, 2, 3, 2, 9
B = 16384
C = W * L
NC, NS, LN = 2, 16, 16
NW = NC * NS
RPW = B // NW
Q = 128
NQ = RPW // Q
NGQ = Q // LN

_OG, _OC, _OHW, _OHB, _OTW, _OTB = 0, 96, 192, 288, 384, 480


def _body(state_hbm, pk_hbm, out_hbm, sv, pv, ov, dp, sem, semo):
    wid = lax.axis_index("s") * NC + lax.axis_index("c")
    base = wid * RPW

    def start(qi, off):
        pltpu.async_copy(state_hbm.at[pl.ds(base + qi * Q, Q)],
                         sv.at[pl.ds(off, Q)], sem)

    start(0, 0)
    pltpu.sync_copy(pk_hbm, pv)

    mi = jnp.arange(LN, dtype=jnp.int32)

    def pick(off):
        return plsc.load_gather(pv, [mi * 6 + off])

    g = [[1.0 / (1.0 + jnp.exp(-pick(_OG + j * L + l))) for l in range(L)]
         for j in range(J)]
    c = [[pick(_OC + j * L + l) for l in range(L)] for j in range(J)]
    hw = [[pick(_OHW + j * I + i) for i in range(I)] for j in range(J)]
    hb = [[pick(_OHB + j * I + i) for i in range(I)] for j in range(J)]
    tw = [[pick(_OTW + lp * I + i) for i in range(I)] for lp in range(L)]
    tb = [plsc.load_gather(pv, [mi * L + (_OTB + lp)]) for lp in range(L)]

    a = [(1.0 - g[0][l]) + (1.0 - g[1][l]) for l in range(L)]
    nb = [-2.0 * ((1.0 - g[0][l]) * c[0][l] + (1.0 - g[1][l]) * c[1][l])
          for l in range(L)]
    cc = sum((1.0 - g[j][l]) * c[j][l] * c[j][l]
             for j in range(J) for l in range(L))
    Rm = [[sum(tw[lp][i] * hw[j][i] for i in range(I)) for j in range(J)]
          for lp in range(L)]
    Sm = [[sum(tw[lp][i] * hb[j][i] for i in range(I)) for j in range(J)]
          for lp in range(L)]
    P = [[sum(Rm[lp][j] * g[j][l] for j in range(J)) for l in range(L)]
         for lp in range(L)]
    Gj = [g[j][0] + g[j][1] for j in range(J)]
    q = [sum(Sm[lp][j] * Gj[j] for j in range(J)) + tb[lp] for lp in range(L)]

    derived = [a[0], a[1], nb[0], nb[1], cc,
               P[0][0], P[0][1], P[1][0], P[1][1], q[0], q[1]]
    for k, vec in enumerate(derived):
        for m in range(M):
            dp[pl.ds((k * M + m) * LN, LN)] = jnp.broadcast_to(vec[m], (LN,))

    def dpl(k, m):
        return dp[pl.ds((k * M + m) * LN, LN)]

    def quarter_chunk(qi, carry):
        pltpu.make_async_copy(state_hbm.at[pl.ds(0, Q)],
                              sv.at[pl.ds(0, Q)], sem).wait()
        dyn = (qi % 2) * Q
        nxt = ((qi + 1) % 2) * Q

        @pl.when(qi < NQ - 1)
        def _():
            start(qi + 1, nxt)

        @pl.when(qi == 1)
        def _():
            pltpu.async_copy(ov.at[pl.ds(0, Q)],
                             out_hbm.at[pl.ds(base, Q)], semo)

        @plsc.parallel_loop(0, NGQ)
        def group(gi):
            rows = mi + gi * LN
            s = [[plsc.load_gather(sv, [dyn + rows,
                                        jnp.full((LN,), w * L + l, jnp.int32)])
                  for l in range(L)] for w in range(W)]
            def rule(m, ow):
                a0, a1 = dpl(0, m), dpl(1, m)
                nb0, nb1 = dpl(2, m), dpl(3, m)
                ccm = dpl(4, m)
                p00, p01 = dpl(5, m), dpl(6, m)
                p10, p11 = dpl(7, m), dpl(8, m)
                q0, q1 = dpl(9, m), dpl(10, m)
                mn = s[0][0] * (a0 * s[0][0] + nb0) + \
                     s[0][1] * (a1 * s[0][1] + nb1) + ccm
                sb0, sb1 = s[0][0], s[0][1]
                for w in range(1, W):
                    t = s[w][0] * (a0 * s[w][0] + nb0) + \
                        s[w][1] * (a1 * s[w][1] + nb1) + ccm
                    lt = t < mn
                    mn = jnp.where(lt, t, mn)
                    sb0 = jnp.where(lt, s[w][0], sb0)
                    sb1 = jnp.where(lt, s[w][1], sb1)
                cl0 = p00 * sb0 + p01 * sb1 + q0
                cl1 = p10 * sb0 + p11 * sb1 + q1
                conf = jnp.exp(-mn)
                return tuple(
                    ow[w] + conf * jnp.exp(
                        -((cl0 - s[w][0]) * (cl0 - s[w][0])
                          + (cl1 - s[w][1]) * (cl1 - s[w][1])))
                    for w in range(W))

            ow = lax.fori_loop(
                0, M, rule, tuple(jnp.zeros((LN,), jnp.float32)
                                  for _ in range(W)), unroll=2)
            orow = rows + qi * Q
            for w in range(W):
                plsc.store_scatter(ov, [orow, jnp.full((LN,), w, jnp.int32)],
                                   ow[w])
        return carry

    lax.fori_loop(0, NQ, quarter_chunk, 0)
    pltpu.make_async_copy(ov.at[pl.ds(0, Q)],
                          out_hbm.at[pl.ds(base, Q)], semo).wait()
    pltpu.sync_copy(ov.at[pl.ds(Q, 3 * Q)],
                    out_hbm.at[pl.ds(base + Q, 3 * Q)])


@jax.jit
def kernel(state, constants, gammas, head_w, head_b, tail_w, tail_b):
    pk = jnp.concatenate([
        gammas.reshape(-1), constants.reshape(-1),
        head_w.reshape(-1), head_b.reshape(-1),
        tail_w.reshape(-1), tail_b.reshape(-1),
    ]).astype(jnp.float32)
    mesh = plsc.VectorSubcoreMesh(
        core_axis_name="c", subcore_axis_name="s",
        num_cores=NC, num_subcores=NS)
    f = pl.kernel(
        _body,
        out_type=jax.ShapeDtypeStruct((B, W), jnp.float32),
        mesh=mesh,
        compiler_params=pltpu.CompilerParams(needs_layout_passes=False),
        scratch_types=[
            pltpu.VMEM((2 * Q, C), jnp.float32),
            pltpu.VMEM((512,), jnp.float32),
            pltpu.VMEM((RPW, W), jnp.float32),
            pltpu.VMEM((11 * M * LN,), jnp.float32),
            pltpu.SemaphoreType.DMA,
            pltpu.SemaphoreType.DMA,
        ],
    )
    return f(state, pk)


# speedup vs baseline: 1.0102x; 1.0102x over previous
import jax
import jax.numpy as jnp
from jax import lax
from jax.experimental import pallas as pl
from jax.experimental.pallas import tpu as pltpu
from jax.experimental.pallas import tpu_sc as plsc

M, J, I, L, W = 16, 2, 3, 2, 9
B = 16384
C = W * L
NC, NS, LN = 2, 16, 16
NW = NC * NS
RPW = B // NW
Q = 128
NQ = RPW // Q
NGQ = Q // LN

_OG, _OC, _OHW, _OHB, _OTW, _OTB = 0, 96, 192, 288, 384, 480


def _body(state_hbm, pk_hbm, out_hbm, sv, pv, ov, dp, sem, semo):
    wid = lax.axis_index("s") * NC + lax.axis_index("c")
    base = wid * RPW

    def start(qi, off):
        pltpu.async_copy(state_hbm.at[pl.ds(base + qi * Q, Q)],
                         sv.at[pl.ds(off, Q)], sem)

    start(0, 0)
    pltpu.sync_copy(pk_hbm, pv)

    mi = jnp.arange(LN, dtype=jnp.int32)

    def pick(off):
        return plsc.load_gather(pv, [mi * 6 + off])

    g = [[1.0 / (1.0 + jnp.exp(-pick(_OG + j * L + l))) for l in range(L)]
         for j in range(J)]
    c = [[pick(_OC + j * L + l) for l in range(L)] for j in range(J)]
    hw = [[pick(_OHW + j * I + i) for i in range(I)] for j in range(J)]
    hb = [[pick(_OHB + j * I + i) for i in range(I)] for j in range(J)]
    tw = [[pick(_OTW + lp * I + i) for i in range(I)] for lp in range(L)]
    tb = [plsc.load_gather(pv, [mi * L + (_OTB + lp)]) for lp in range(L)]

    a = [(1.0 - g[0][l]) + (1.0 - g[1][l]) for l in range(L)]
    nb = [-2.0 * ((1.0 - g[0][l]) * c[0][l] + (1.0 - g[1][l]) * c[1][l])
          for l in range(L)]
    cc = sum((1.0 - g[j][l]) * c[j][l] * c[j][l]
             for j in range(J) for l in range(L))
    Rm = [[sum(tw[lp][i] * hw[j][i] for i in range(I)) for j in range(J)]
          for lp in range(L)]
    Sm = [[sum(tw[lp][i] * hb[j][i] for i in range(I)) for j in range(J)]
          for lp in range(L)]
    P = [[sum(Rm[lp][j] * g[j][l] for j in range(J)) for l in range(L)]
         for lp in range(L)]
    Gj = [g[j][0] + g[j][1] for j in range(J)]
    q = [sum(Sm[lp][j] * Gj[j] for j in range(J)) + tb[lp] for lp in range(L)]

    derived = [a[0], a[1], nb[0], nb[1], cc,
               P[0][0], P[0][1], P[1][0], P[1][1], q[0], q[1]]
    for k, vec in enumerate(derived):
        for m in range(M):
            dp[pl.ds((k * M + m) * LN, LN)] = jnp.broadcast_to(vec[m], (LN,))

    def dpl(k, m):
        return dp[pl.ds((k * M + m) * LN, LN)]

    def quarter_chunk(qi, carry):
        pltpu.make_async_copy(state_hbm.at[pl.ds(0, Q)],
                              sv.at[pl.ds(0, Q)], sem).wait()
        dyn = (qi % 2) * Q
        nxt = ((qi + 1) % 2) * Q

        @pl.when(qi < NQ - 1)
        def _():
            start(qi + 1, nxt)

        @pl.when(qi == 1)
        def _():
            pltpu.async_copy(ov.at[pl.ds(0, Q)],
                             out_hbm.at[pl.ds(base, Q)], semo)

        @plsc.parallel_loop(0, NGQ)
        def group(gi):
            rows = mi + gi * LN
            s = [[plsc.load_gather(sv, [dyn + rows,
                                        jnp.full((LN,), w * L + l, jnp.int32)])
                  for l in range(L)] for w in range(W)]
            def rule(m, ow):
                a0, a1 = dpl(0, m), dpl(1, m)
                nb0, nb1 = dpl(2, m), dpl(3, m)
                ccm = dpl(4, m)
                p00, p01 = dpl(5, m), dpl(6, m)
                p10, p11 = dpl(7, m), dpl(8, m)
                q0, q1 = dpl(9, m), dpl(10, m)
                mn = s[0][0] * (a0 * s[0][0] + nb0) + \
                     s[0][1] * (a1 * s[0][1] + nb1) + ccm
                sb0, sb1 = s[0][0], s[0][1]
                for w in range(1, W):
                    t = s[w][0] * (a0 * s[w][0] + nb0) + \
                        s[w][1] * (a1 * s[w][1] + nb1) + ccm
                    lt = t < mn
                    mn = jnp.where(lt, t, mn)
                    sb0 = jnp.where(lt, s[w][0], sb0)
                    sb1 = jnp.where(lt, s[w][1], sb1)
                cl0 = p00 * sb0 + p01 * sb1 + q0
                cl1 = p10 * sb0 + p11 * sb1 + q1
                conf = jnp.exp(-mn)
                return tuple(
                    ow[w] + conf * jnp.exp(
                        -((cl0 - s[w][0]) * (cl0 - s[w][0])
                          + (cl1 - s[w][1]) * (cl1 - s[w][1])))
                    for w in range(W))

            ow = lax.fori_loop(
                0, M, rule, tuple(jnp.zeros((LN,), jnp.float32)
                                  for _ in range(W)))
            orow = rows + qi * Q
            for w in range(W):
                plsc.store_scatter(ov, [orow, jnp.full((LN,), w, jnp.int32)],
                                   ow[w])
        return carry

    lax.fori_loop(0, NQ, quarter_chunk, 0)
    pltpu.make_async_copy(ov.at[pl.ds(0, Q)],
                          out_hbm.at[pl.ds(base, Q)], semo).wait()
    pltpu.sync_copy(ov.at[pl.ds(Q, 3 * Q)],
                    out_hbm.at[pl.ds(base + Q, 3 * Q)])


@jax.jit
def kernel(state, constants, gammas, head_w, head_b, tail_w, tail_b):
    pk = jnp.concatenate([
        gammas.reshape(-1), constants.reshape(-1),
        head_w.reshape(-1), head_b.reshape(-1),
        tail_w.reshape(-1), tail_b.reshape(-1),
    ]).astype(jnp.float32)
    mesh = plsc.VectorSubcoreMesh(
        core_axis_name="c", subcore_axis_name="s",
        num_cores=NC, num_subcores=NS)
    f = pl.kernel(
        _body,
        out_type=jax.ShapeDtypeStruct((B, W), jnp.float32),
        mesh=mesh,
        compiler_params=pltpu.CompilerParams(needs_layout_passes=False),
        scratch_types=[
            pltpu.VMEM((2 * Q, C), jnp.float32),
            pltpu.VMEM((512,), jnp.float32),
            pltpu.VMEM((RPW, W), jnp.float32),
            pltpu.VMEM((11 * M * LN,), jnp.float32),
            pltpu.SemaphoreType.DMA,
            pltpu.SemaphoreType.DMA,
        ],
    )
    return f(state, pk)


# speedup vs baseline: 1.4320x; 1.4176x over previous
import jax
import jax.numpy as jnp
from jax import lax
from jax.experimental import pallas as pl
from jax.experimental.pallas import tpu as pltpu
from jax.experimental.pallas import tpu_sc as plsc

M, J, I, L, W = 16, 2, 3, 2, 9
B = 16384
C = W * L
NC, NS, LN = 2, 16, 16
NW = NC * NS
RPW = B // NW
Q = 128
NQ = RPW // Q
NGQ = Q // LN

_OG, _OC, _OHW, _OHB, _OTW, _OTB = 0, 96, 192, 288, 384, 480


def _body(state_hbm, pk_hbm, out_hbm, sv, pv, ov, dp, sem, semo):
    wid = lax.axis_index("s") * NC + lax.axis_index("c")
    base = wid * RPW

    def start(qi, off):
        pltpu.async_copy(state_hbm.at[:, pl.ds(base + qi * Q, Q)],
                         sv.at[:, pl.ds(off, Q)], sem)

    start(0, 0)
    pltpu.sync_copy(pk_hbm, pv)

    mi = jnp.arange(LN, dtype=jnp.int32)

    def pick(off):
        return plsc.load_gather(pv, [mi * 6 + off])

    g = [[1.0 / (1.0 + jnp.exp(-pick(_OG + j * L + l))) for l in range(L)]
         for j in range(J)]
    c = [[pick(_OC + j * L + l) for l in range(L)] for j in range(J)]
    hw = [[pick(_OHW + j * I + i) for i in range(I)] for j in range(J)]
    hb = [[pick(_OHB + j * I + i) for i in range(I)] for j in range(J)]
    tw = [[pick(_OTW + lp * I + i) for i in range(I)] for lp in range(L)]
    tb = [plsc.load_gather(pv, [mi * L + (_OTB + lp)]) for lp in range(L)]

    a = [(1.0 - g[0][l]) + (1.0 - g[1][l]) for l in range(L)]
    nb = [-2.0 * ((1.0 - g[0][l]) * c[0][l] + (1.0 - g[1][l]) * c[1][l])
          for l in range(L)]
    cc = sum((1.0 - g[j][l]) * c[j][l] * c[j][l]
             for j in range(J) for l in range(L))
    Rm = [[sum(tw[lp][i] * hw[j][i] for i in range(I)) for j in range(J)]
          for lp in range(L)]
    Sm = [[sum(tw[lp][i] * hb[j][i] for i in range(I)) for j in range(J)]
          for lp in range(L)]
    P = [[sum(Rm[lp][j] * g[j][l] for j in range(J)) for l in range(L)]
         for lp in range(L)]
    Gj = [g[j][0] + g[j][1] for j in range(J)]
    q = [sum(Sm[lp][j] * Gj[j] for j in range(J)) + tb[lp] for lp in range(L)]

    derived = [a[0], a[1], nb[0], nb[1], cc,
               P[0][0], P[0][1], P[1][0], P[1][1], q[0], q[1]]
    for k, vec in enumerate(derived):
        for m in range(M):
            dp[pl.ds((k * M + m) * LN, LN)] = jnp.broadcast_to(vec[m], (LN,))

    def dpl(k, m):
        return dp[pl.ds((k * M + m) * LN, LN)]

    def quarter_chunk(qi, carry):
        pltpu.make_async_copy(state_hbm.at[:, pl.ds(0, Q)],
                              sv.at[:, pl.ds(0, Q)], sem).wait()
        dyn = (qi % 2) * Q
        nxt = ((qi + 1) % 2) * Q

        @pl.when(qi < NQ - 1)
        def _():
            start(qi + 1, nxt)

        @pl.when(qi == 1)
        def _():
            pltpu.async_copy(ov.at[:, pl.ds(0, Q)],
                             out_hbm.at[:, pl.ds(base, Q)], semo)

        @plsc.parallel_loop(0, NGQ)
        def group(gi):
            col = dyn + gi * LN
            s = [[sv[w * L + l, pl.ds(col, LN)]
                  for l in range(L)] for w in range(W)]
            def rule(m, ow):
                a0, a1 = dpl(0, m), dpl(1, m)
                nb0, nb1 = dpl(2, m), dpl(3, m)
                ccm = dpl(4, m)
                p00, p01 = dpl(5, m), dpl(6, m)
                p10, p11 = dpl(7, m), dpl(8, m)
                q0, q1 = dpl(9, m), dpl(10, m)
                mn = s[0][0] * (a0 * s[0][0] + nb0) + \
                     s[0][1] * (a1 * s[0][1] + nb1) + ccm
                sb0, sb1 = s[0][0], s[0][1]
                for w in range(1, W):
                    t = s[w][0] * (a0 * s[w][0] + nb0) + \
                        s[w][1] * (a1 * s[w][1] + nb1) + ccm
                    lt = t < mn
                    mn = jnp.where(lt, t, mn)
                    sb0 = jnp.where(lt, s[w][0], sb0)
                    sb1 = jnp.where(lt, s[w][1], sb1)
                cl0 = p00 * sb0 + p01 * sb1 + q0
                cl1 = p10 * sb0 + p11 * sb1 + q1
                conf = jnp.exp(-mn)
                return tuple(
                    ow[w] + conf * jnp.exp(
                        -((cl0 - s[w][0]) * (cl0 - s[w][0])
                          + (cl1 - s[w][1]) * (cl1 - s[w][1])))
                    for w in range(W))

            ow = lax.fori_loop(
                0, M, rule, tuple(jnp.zeros((LN,), jnp.float32)
                                  for _ in range(W)))
            ocol = qi * Q + gi * LN
            for w in range(W):
                ov[w, pl.ds(ocol, LN)] = ow[w]
        return carry

    lax.fori_loop(0, NQ, quarter_chunk, 0)
    pltpu.make_async_copy(ov.at[:, pl.ds(0, Q)],
                          out_hbm.at[:, pl.ds(base, Q)], semo).wait()
    pltpu.sync_copy(ov.at[:, pl.ds(Q, 3 * Q)],
                    out_hbm.at[:, pl.ds(base + Q, 3 * Q)])


@jax.jit
def kernel(state, constants, gammas, head_w, head_b, tail_w, tail_b):
    pk = jnp.concatenate([
        gammas.reshape(-1), constants.reshape(-1),
        head_w.reshape(-1), head_b.reshape(-1),
        tail_w.reshape(-1), tail_b.reshape(-1),
    ]).astype(jnp.float32)
    mesh = plsc.VectorSubcoreMesh(
        core_axis_name="c", subcore_axis_name="s",
        num_cores=NC, num_subcores=NS)
    f = pl.kernel(
        _body,
        out_type=jax.ShapeDtypeStruct((W, B), jnp.float32),
        mesh=mesh,
        compiler_params=pltpu.CompilerParams(needs_layout_passes=False),
        scratch_types=[
            pltpu.VMEM((C, 2 * Q), jnp.float32),
            pltpu.VMEM((512,), jnp.float32),
            pltpu.VMEM((W, RPW), jnp.float32),
            pltpu.VMEM((11 * M * LN,), jnp.float32),
            pltpu.SemaphoreType.DMA,
            pltpu.SemaphoreType.DMA,
        ],
    )
    return f(state.T, pk).T
